# Initial kernel scaffold; baseline (speedup 1.0000x reference)
import functools

import jax
import jax.numpy as jnp
from jax import lax
from jax.experimental import pallas as pl
from jax.experimental.pallas import tpu as pltpu
from jax.experimental.pallas import tpu_sc as plsc

N_EMBED = 1000000
D_EMBED = 16
BATCH = 16384
HIST = 200

TOTAL = BATCH * HIST
NUM_WORKERS = 32
PER_WORKER = TOTAL // NUM_WORKERS
CHUNK = 2048
NCHUNK = PER_WORKER // CHUNK

_mesh = plsc.VectorSubcoreMesh(core_axis_name="c", subcore_axis_name="s")


@functools.partial(
    pl.kernel,
    mesh=_mesh,
    out_type=jax.ShapeDtypeStruct((TOTAL, D_EMBED), jnp.float32),
    scratch_types=[
        pltpu.VMEM((CHUNK,), jnp.int32),
        pltpu.VMEM((CHUNK, D_EMBED), jnp.float32),
        pltpu.SemaphoreType.DMA,
    ],
)
def _gather_kernel(idx_hbm, table_hbm, out_hbm, idx_v, rows_v, sem):
    wid = lax.axis_index("s") * 2 + lax.axis_index("c")
    base = wid * PER_WORKER

    def body(c, carry):
        off = base + c * CHUNK
        pltpu.sync_copy(idx_hbm.at[pl.ds(off, CHUNK)], idx_v)
        pltpu.async_copy(table_hbm.at[idx_v], rows_v, sem).wait()
        pltpu.sync_copy(rows_v, out_hbm.at[pl.ds(off, CHUNK)])
        return carry

    lax.fori_loop(0, NCHUNK, body, 0)


def kernel(x, table):
    idx = jnp.asarray(x, jnp.int32).reshape(TOTAL)
    out = _gather_kernel(idx, table)
    return out.reshape(BATCH, HIST, D_EMBED)


# speedup vs baseline: 2.4903x; 2.4903x over previous
import functools

import jax
import jax.numpy as jnp
from jax import lax
from jax.experimental import pallas as pl
from jax.experimental.pallas import tpu as pltpu
from jax.experimental.pallas import tpu_sc as plsc

N_EMBED = 1000000
D_EMBED = 16
BATCH = 16384
HIST = 200

TOTAL = BATCH * HIST
NUM_WORKERS = 32
PER_WORKER = TOTAL // NUM_WORKERS
CHUNK = 2048
NCHUNK = PER_WORKER // CHUNK

_mesh = plsc.VectorSubcoreMesh(core_axis_name="c", subcore_axis_name="s")


@functools.partial(
    pl.kernel,
    mesh=_mesh,
    out_type=jax.ShapeDtypeStruct((TOTAL, D_EMBED), jnp.float32),
    scratch_types=[
        pltpu.VMEM((CHUNK,), jnp.int32),
        pltpu.VMEM((CHUNK, D_EMBED), jnp.float32),
        pltpu.SemaphoreType.DMA,
    ],
    compiler_params=pltpu.CompilerParams(use_tc_tiling_on_sc=False),
)
def _gather_kernel(idx_hbm, table_hbm, out_hbm, idx_v, rows_v, sem):
    wid = lax.axis_index("s") * 2 + lax.axis_index("c")
    base = wid * PER_WORKER

    def body(c, carry):
        off = base + c * CHUNK
        pltpu.sync_copy(idx_hbm.at[pl.ds(off, CHUNK)], idx_v)
        pltpu.async_copy(table_hbm.at[idx_v], rows_v, sem).wait()
        pltpu.sync_copy(rows_v, out_hbm.at[pl.ds(off, CHUNK)])
        return carry

    lax.fori_loop(0, NCHUNK, body, 0)


def kernel(x, table):
    idx = jnp.asarray(x, jnp.int32).reshape(TOTAL)
    out = _gather_kernel(idx, table)
    return out.reshape(BATCH, HIST, D_EMBED)


# speedup vs baseline: 2.5407x; 1.0202x over previous
import functools

import jax
import jax.numpy as jnp
from jax import lax
from jax.experimental import pallas as pl
from jax.experimental.pallas import tpu as pltpu
from jax.experimental.pallas import tpu_sc as plsc

N_EMBED = 1000000
D_EMBED = 16
BATCH = 16384
HIST = 200

TOTAL = BATCH * HIST
NUM_WORKERS = 32
PER_WORKER = TOTAL // NUM_WORKERS
CHUNK = 2560
NCHUNK = PER_WORKER // CHUNK
NBUF = 2

_mesh = plsc.VectorSubcoreMesh(core_axis_name="c", subcore_axis_name="s")


@functools.partial(
    pl.kernel,
    mesh=_mesh,
    out_type=jax.ShapeDtypeStruct((TOTAL, D_EMBED), jnp.float32),
    scratch_types=[
        pltpu.VMEM((NBUF, CHUNK), jnp.int32),
        pltpu.VMEM((NBUF, CHUNK, D_EMBED), jnp.float32),
        pltpu.SemaphoreType.DMA,
        pltpu.SemaphoreType.DMA,
        pltpu.SemaphoreType.DMA,
        pltpu.SemaphoreType.DMA,
        pltpu.SemaphoreType.DMA,
    ],
    compiler_params=pltpu.CompilerParams(use_tc_tiling_on_sc=False),
)
def _gather_kernel(idx_hbm, table_hbm, out_hbm, idx_v, rows_v,
                   sem_g, si0, si1, so0, so1):
    sem_idx = (si0, si1)
    sem_out = (so0, so1)
    wid = lax.axis_index("s") * 2 + lax.axis_index("c")
    base = wid * PER_WORKER

    def issue_idx(c, b):
        pltpu.async_copy(idx_hbm.at[pl.ds(base + c * CHUNK, CHUNK)],
                         idx_v.at[b], sem_idx[b])

    def wait_idx(c, b):
        pltpu.make_async_copy(idx_hbm.at[pl.ds(base + c * CHUNK, CHUNK)],
                              idx_v.at[b], sem_idx[b]).wait()

    def gather(b):
        pltpu.async_copy(table_hbm.at[idx_v.at[b]], rows_v.at[b], sem_g).wait()

    def issue_out(c, b):
        pltpu.async_copy(rows_v.at[b],
                         out_hbm.at[pl.ds(base + c * CHUNK, CHUNK)], sem_out[b])

    def wait_out(c, b):
        pltpu.make_async_copy(rows_v.at[b],
                              out_hbm.at[pl.ds(base + c * CHUNK, CHUNK)],
                              sem_out[b]).wait()

    issue_idx(0, 0)
    issue_idx(1, 1)
    for c in range(NBUF):
        b = c
        wait_idx(c, b)
        gather(b)
        issue_out(c, b)
        issue_idx(c + NBUF, b)

    def body(i, carry):
        c0 = NBUF + NBUF * i
        for b in range(NBUF):
            c = c0 + b
            wait_out(c - NBUF, b)
            wait_idx(c, b)
            gather(b)
            issue_out(c, b)
            issue_idx(c + NBUF, b)
        return carry

    lax.fori_loop(0, (NCHUNK - 2 * NBUF) // NBUF, body, 0)

    for c in range(NCHUNK - NBUF, NCHUNK):
        b = c % NBUF
        wait_out(c - NBUF, b)
        wait_idx(c, b)
        gather(b)
        issue_out(c, b)
    for c in range(NCHUNK - NBUF, NCHUNK):
        wait_out(c, c % NBUF)


def kernel(x, table):
    idx = jnp.asarray(x, jnp.int32).reshape(TOTAL)
    out = _gather_kernel(idx, table)
    return out.reshape(BATCH, HIST, D_EMBED)


# speedup vs baseline: 2.5721x; 1.0124x over previous
import functools

import jax
import jax.numpy as jnp
from jax import lax
from jax.experimental import pallas as pl
from jax.experimental.pallas import tpu as pltpu
from jax.experimental.pallas import tpu_sc as plsc

N_EMBED = 1000000
D_EMBED = 16
BATCH = 16384
HIST = 200

TOTAL = BATCH * HIST
NUM_WORKERS = 32
PER_WORKER = TOTAL // NUM_WORKERS
CHUNK = 1600
NCHUNK = PER_WORKER // CHUNK
NBUF = 4
LAG = 2

_mesh = plsc.VectorSubcoreMesh(core_axis_name="c", subcore_axis_name="s")


@functools.partial(
    pl.kernel,
    mesh=_mesh,
    out_type=jax.ShapeDtypeStruct((TOTAL, D_EMBED), jnp.float32),
    scratch_types=[
        pltpu.VMEM((NBUF, CHUNK), jnp.int32),
        pltpu.VMEM((NBUF, CHUNK, D_EMBED), jnp.float32),
    ] + [pltpu.SemaphoreType.DMA] * (3 * NBUF),
    compiler_params=pltpu.CompilerParams(use_tc_tiling_on_sc=False),
)
def _gather_kernel(idx_hbm, table_hbm, out_hbm, idx_v, rows_v, *sems):
    sem_idx = sems[:NBUF]
    sem_g = sems[NBUF:2 * NBUF]
    sem_out = sems[2 * NBUF:]
    wid = lax.axis_index("s") * 2 + lax.axis_index("c")
    base = wid * PER_WORKER

    def issue_idx(c, b):
        pltpu.async_copy(idx_hbm.at[pl.ds(base + c * CHUNK, CHUNK)],
                         idx_v.at[b], sem_idx[b])

    def wait_idx(c, b):
        pltpu.make_async_copy(idx_hbm.at[pl.ds(base + c * CHUNK, CHUNK)],
                              idx_v.at[b], sem_idx[b]).wait()

    def issue_gather(b):
        pltpu.async_copy(table_hbm.at[idx_v.at[b]], rows_v.at[b], sem_g[b])

    def wait_gather(b):
        pltpu.make_async_copy(table_hbm.at[idx_v.at[b]], rows_v.at[b],
                              sem_g[b]).wait()

    def issue_out(c, b):
        pltpu.async_copy(rows_v.at[b],
                         out_hbm.at[pl.ds(base + c * CHUNK, CHUNK)], sem_out[b])

    def wait_out(c, b):
        pltpu.make_async_copy(rows_v.at[b],
                              out_hbm.at[pl.ds(base + c * CHUNK, CHUNK)],
                              sem_out[b]).wait()

    def step(c, b, do_wait_out, do_drain, do_prefetch):
        if do_wait_out:
            wait_out(c - NBUF, b)
        wait_idx(c, b)
        issue_gather(b)
        if do_drain:
            cb = c - LAG
            bb = cb % NBUF if isinstance(cb, int) else (b - LAG) % NBUF
            wait_gather(bb)
            issue_out(cb, bb)
            if do_prefetch:
                issue_idx(cb + NBUF, bb)

    for c in range(NBUF):
        issue_idx(c, c)

    for c in range(NBUF):
        step(c, c, False, c >= LAG, c - LAG + NBUF < NCHUNK)

    steady_end = NCHUNK - NBUF + LAG
    m = (steady_end - NBUF) // NBUF

    def body(i, carry):
        c0 = NBUF + NBUF * i
        for b in range(NBUF):
            step(c0 + b, b, True, True, True)
        return carry

    if m > 0:
        lax.fori_loop(0, m, body, 0)

    for c in range(NBUF + m * NBUF, NCHUNK):
        step(c, c % NBUF, True, c >= LAG, c - LAG + NBUF < NCHUNK)

    for cb in range(NCHUNK - LAG, NCHUNK):
        bb = cb % NBUF
        wait_gather(bb)
        issue_out(cb, bb)

    for c in range(NCHUNK - NBUF, NCHUNK):
        wait_out(c, c % NBUF)


def kernel(x, table):
    idx = jnp.asarray(x, jnp.int32).reshape(TOTAL)
    out = _gather_kernel(idx, table)
    return out.reshape(BATCH, HIST, D_EMBED)


# speedup vs baseline: 9.4322x; 3.6671x over previous
import functools

import jax
import jax.numpy as jnp
from jax import lax
from jax.experimental import pallas as pl
from jax.experimental.pallas import tpu as pltpu
from jax.experimental.pallas import tpu_sc as plsc

N_EMBED = 1000000
D_EMBED = 16
BATCH = 16384
HIST = 200

TOTAL = BATCH * HIST
NUM_WORKERS = 32
KB = 1024
UNITS_PER_H = BATCH // KB
NUNIT = TOTAL // KB
PER_WORKER = NUNIT // NUM_WORKERS
NBUF = 3
UNROLL = 16
BLK = 8 * KB

_mesh = plsc.VectorSubcoreMesh(core_axis_name="c", subcore_axis_name="s")


@functools.partial(
    pl.kernel,
    mesh=_mesh,
    out_type=jax.ShapeDtypeStruct((TOTAL * D_EMBED,), jnp.float32),
    scratch_types=[
        pltpu.VMEM((NBUF, KB), jnp.int32),
        pltpu.VMEM((NBUF, KB, D_EMBED), jnp.float32),
        pltpu.VMEM((NBUF, 2 * BLK), jnp.float32),
    ] + [pltpu.SemaphoreType.DMA] * (3 * NBUF),
    compiler_params=pltpu.CompilerParams(use_tc_tiling_on_sc=False,
                                         needs_layout_passes=False),
)
def _gather_kernel(idx_hbm, table_hbm, out_hbm, idx_v, rows_v, trans_v, *sems):
    sem_idx = sems[:NBUF]
    sem_g = sems[NBUF:2 * NBUF]
    sem_out = sems[2 * NBUF:]
    wid = lax.axis_index("s") * 2 + lax.axis_index("c")
    g0 = wid * PER_WORKER
    lane = lax.iota(jnp.int32, 16)
    lane_block = (lane // 8) * BLK + (lane % 8) * 128

    def issue_idx(c, b):
        pltpu.async_copy(idx_hbm.at[pl.ds((g0 + c) * KB, KB)],
                         idx_v.at[b], sem_idx[b])

    def wait_idx(c, b):
        pltpu.make_async_copy(idx_hbm.at[pl.ds((g0 + c) * KB, KB)],
                              idx_v.at[b], sem_idx[b]).wait()

    def issue_gather(b):
        pltpu.async_copy(table_hbm.at[idx_v.at[b]], rows_v.at[b], sem_g[b])

    def wait_gather(b):
        pltpu.make_async_copy(table_hbm.at[idx_v.at[b]], rows_v.at[b],
                              sem_g[b]).wait()

    rot = [(lane + j) % 16 for j in range(UNROLL)]
    wbase = [lane_block + rot[j] for j in range(UNROLL)]

    def transpose(b):
        rows = rows_v.at[b]
        trans = trans_v.at[b]

        @plsc.parallel_loop(0, KB // UNROLL, unroll=4)
        def tbody(t):
            r0 = t * UNROLL
            pos0 = (r0 // 128) * 1024 + (r0 % 128)
            for j in range(UNROLL):
                v = plsc.load_gather(rows, [r0 + rot[j], lane])
                plsc.store_scatter(trans, [wbase[j] + pos0], v)

    def out_slices(c, b):
        g = g0 + c
        h = g // UNITS_PER_H
        q = g % UNITS_PER_H
        for s in range(2):
            off = h * (D_EMBED * BATCH) + s * (BATCH * 8) + q * BLK
            yield (trans_v.at[b, pl.ds(s * BLK, BLK)],
                   out_hbm.at[pl.ds(off, BLK)])

    def issue_outs(c, b):
        for src, dst in out_slices(c, b):
            pltpu.async_copy(src, dst, sem_out[b])

    def wait_outs(c, b):
        for src, dst in out_slices(c, b):
            pltpu.make_async_copy(src, dst, sem_out[b]).wait()

    def step(c, b, do_wait_out, do_drain, do_prefetch):
        if do_wait_out:
            wait_outs(c - NBUF, b)
        wait_idx(c, b)
        issue_gather(b)
        if do_drain:
            cb = c - 1
            bb = cb % NBUF if isinstance(cb, int) else (b - 1) % NBUF
            wait_gather(bb)
            if do_prefetch:
                issue_idx(cb + NBUF, bb)
            transpose(bb)
            issue_outs(cb, bb)

    for c in range(NBUF):
        issue_idx(c, c)
    for c in range(NBUF):
        step(c, c, False, c >= 1, c - 1 + NBUF < PER_WORKER)

    steady_end = PER_WORKER - NBUF + 1
    m = (steady_end - NBUF) // NBUF

    def body(i, carry):
        c0 = NBUF + NBUF * i
        for b in range(NBUF):
            step(c0 + b, b, True, True, True)
        return carry

    if m > 0:
        lax.fori_loop(0, m, body, 0)

    for c in range(NBUF + m * NBUF, PER_WORKER):
        step(c, c % NBUF, True, c >= 1, c - 1 + NBUF < PER_WORKER)

    bb = (PER_WORKER - 1) % NBUF
    wait_gather(bb)
    transpose(bb)
    issue_outs(PER_WORKER - 1, bb)
    for c in range(PER_WORKER - NBUF, PER_WORKER):
        wait_outs(c, c % NBUF)


def kernel(x, table):
    idx = jnp.transpose(x).reshape(TOTAL)
    flat = _gather_kernel(jnp.asarray(idx, jnp.int32), table)
    a = flat.reshape(HIST, 2, BATCH // 128, 8, 128)
    out_t = a.transpose(0, 1, 3, 2, 4).reshape(HIST, D_EMBED, BATCH)
    return out_t.transpose(2, 0, 1)
